# baseline (speedup 1.0000x reference)
import jax
import jax.numpy as jnp
from jax.experimental import pallas as pl


def _copy_block(x_ref, o_ref):
    o_ref[...] = x_ref[...]


def kernel(features, labels):
    del labels
    n, h, w = features.shape
    rows, cols = n, h * w
    flat = features.reshape(rows, cols)
    block_rows = 512
    out = pl.pallas_call(
        _copy_block,
        grid=(rows // block_rows,),
        in_specs=[pl.BlockSpec((block_rows, cols), lambda i: (i, 0))],
        out_specs=pl.BlockSpec((block_rows, cols), lambda i: (i, 0)),
        out_shape=jax.ShapeDtypeStruct((rows, cols), features.dtype),
    )(flat)
    return out.reshape(n, h, w)


# speedup vs baseline: 3.7736x; 3.7736x over previous
import jax
import jax.numpy as jnp
from jax.experimental import pallas as pl


def _copy_block(x_ref, o_ref):
    o_ref[...] = x_ref[...]


def kernel(features, labels):
    del labels
    n, h, w = features.shape
    rows, cols = n * h, w
    flat = features.reshape(rows, cols)
    block_rows = 16384
    out = pl.pallas_call(
        _copy_block,
        grid=(rows // block_rows,),
        in_specs=[pl.BlockSpec((block_rows, cols), lambda i: (i, 0))],
        out_specs=pl.BlockSpec((block_rows, cols), lambda i: (i, 0)),
        out_shape=jax.ShapeDtypeStruct((rows, cols), features.dtype),
    )(flat)
    return out.reshape(n, h, w)
